# baseline (speedup 1.0000x reference)
import jax
import jax.numpy as jnp
from jax.experimental import pallas as pl

_K = 64
_BLK = 64
_TARGETS = (70, 65, 64, 63)


def _select_kernel(x_ref, w_ref, op_ref, wm_ref):
    x = x_ref[...]
    w = w_ref[...]
    blk, n = x.shape

    s = jax.lax.bitcast_convert_type(x, jnp.int32)
    key = jnp.where(s < 0, s ^ jnp.int32(0x7FFFFFFF), s)
    sw = jax.lax.bitcast_convert_type(w, jnp.int32)
    keyw = jnp.where(sw < 0, sw ^ jnp.int32(0x7FFFFFFF), sw)

    maxk = jnp.max(key.reshape(blk, n // 128, 128), axis=1)
    lmax = jnp.max(maxk, axis=1, keepdims=True)
    lmin = jnp.min(maxk, axis=1, keepdims=True)

    def cbody(_, c):
        lo_c, hi_c = c
        mid = (lo_c & hi_c) + ((lo_c ^ hi_c) >> 1)
        cnt = jnp.sum((maxk >= mid).astype(jnp.int32), axis=1, keepdims=True)
        ge = cnt >= _K
        return jnp.where(ge, mid, lo_c), jnp.where(ge, hi_c, mid)

    c64, _ = jax.lax.fori_loop(0, 32, cbody, (lmin, lmax + 1))

    lo0 = c64
    hi0 = jnp.maximum(lmax, keyw) + 1
    cl0 = (jnp.sum((key >= lo0).astype(jnp.int32), axis=1, keepdims=True)
           + (keyw >= lo0).astype(jnp.int32))
    ch0 = jnp.zeros_like(cl0)

    def _done(lo, hi, cl, ch):
        return (hi <= lo + 1) | (cl == _K) | (ch == _K - 1)

    def vcond(c):
        lo, hi, cl, ch = c
        return jnp.any(~_done(lo, hi, cl, ch))

    def vbody(c):
        lo, hi, cl, ch = c
        span = hi - lo
        mid_bis = (lo & hi) + ((lo ^ hi) >> 1)
        bnd = jnp.maximum(span >> 3, 1)
        lo_f = lo.astype(jnp.float32)
        span_f = span.astype(jnp.float32)
        denom = jnp.maximum(cl - ch, 1).astype(jnp.float32)
        mids = []
        for t in _TARGETS:
            frac = jnp.clip((cl - t).astype(jnp.float32) / denom, 0.0, 1.0)
            mid_f = lo_f + frac * span_f
            mid_i = jnp.clip(mid_f.astype(jnp.int32), lo + bnd, hi - bnd)
            mids.append(jnp.where(span > 0, mid_i, mid_bis))
        cnts = [(jnp.sum((key >= m).astype(jnp.int32), axis=1, keepdims=True)
                 + (keyw >= m).astype(jnp.int32)) for m in mids]
        upd = ~_done(lo, hi, cl, ch)
        for m, cnt in zip(mids, cnts):
            ge = (cnt >= _K) & upd & (m > lo) & (m < hi)
            lt = (cnt < _K) & upd & (m > lo) & (m < hi)
            lo = jnp.where(ge, m, lo)
            cl = jnp.where(ge, cnt, cl)
            hi = jnp.where(lt, m, hi)
            ch = jnp.where(lt, cnt, ch)
        return lo, hi, cl, ch

    lo, hi, cl, ch = jax.lax.while_loop(vcond, vbody, (lo0, hi0, cl0, ch0))

    imax = jnp.int32(0x7FFFFFFF)
    imin = jnp.int32(-0x80000000)
    mmin = jnp.min(jnp.where(key >= lo, key, imax), axis=1, keepdims=True)
    mmin = jnp.where(keyw >= lo, jnp.minimum(mmin, keyw), mmin)
    mmax = jnp.max(jnp.where(key < hi, key, imin), axis=1, keepdims=True)
    mmax = jnp.where(keyw < hi, jnp.maximum(mmax, keyw), mmax)

    conv = hi <= lo + 1
    fin_a = cl == _K
    m64 = jnp.where(conv, lo, jnp.where(fin_a, mmin, mmax))

    eqw = keyw == m64
    gtw = keyw > m64
    cnt_eq = (jnp.sum((key == m64).astype(jnp.int32), axis=1, keepdims=True)
              + eqw.astype(jnp.int32))
    cnt_ge = jnp.where(conv, cl, jnp.where(fin_a, _K, ch + cnt_eq))
    cnt_gt = cnt_ge - cnt_eq

    idx = jax.lax.broadcasted_iota(jnp.int32, (blk, n), 1)
    loi0 = jnp.where(cnt_ge == _K, jnp.int32(n), jnp.int32(0))
    hii0 = jnp.full((blk, 1), n + 1, jnp.int32)

    def icond(c):
        lo_i, hi_i = c
        return jnp.any(hi_i - lo_i > 1)

    def ibody(c):
        lo_i, hi_i = c
        mid = lo_i + (hi_i - lo_i) // 2
        f = cnt_gt + jnp.sum(((key == m64) & (idx < mid)).astype(jnp.int32),
                             axis=1, keepdims=True)
        ge = f >= _K
        return jnp.where(ge, lo_i, mid), jnp.where(ge, mid, hi_i)

    _, istar = jax.lax.while_loop(icond, ibody, (loi0, hii0))

    op_ref[...] = (key > m64) | ((key == m64) & (idx < istar))
    wm_ref[...] = gtw | (eqw & (istar == n + 1))


def kernel(slot_scores, write_score):
    b, n = slot_scores.shape
    w2d = write_score.reshape(b, 1)
    grid = b // _BLK
    op_mask, wm2d = pl.pallas_call(
        _select_kernel,
        grid=(grid,),
        in_specs=[
            pl.BlockSpec((_BLK, n), lambda i: (i, 0)),
            pl.BlockSpec((_BLK, 1), lambda i: (i, 0)),
        ],
        out_specs=[
            pl.BlockSpec((_BLK, n), lambda i: (i, 0)),
            pl.BlockSpec((_BLK, 1), lambda i: (i, 0)),
        ],
        out_shape=[
            jax.ShapeDtypeStruct((b, n), jnp.bool_),
            jax.ShapeDtypeStruct((b, 1), jnp.bool_),
        ],
    )(slot_scores, w2d)
    return op_mask, wm2d.reshape(b)


# speedup vs baseline: 1.0503x; 1.0503x over previous
import jax
import jax.numpy as jnp
from jax.experimental import pallas as pl

_K = 64
_BLK = 32
_TARGETS = (70, 65, 64, 63)


def _select_kernel(x_ref, w_ref, op_ref, wm_ref):
    x = x_ref[...]
    w = w_ref[...]
    blk, n = x.shape

    s = jax.lax.bitcast_convert_type(x, jnp.int32)
    key = jnp.where(s < 0, s ^ jnp.int32(0x7FFFFFFF), s)
    sw = jax.lax.bitcast_convert_type(w, jnp.int32)
    keyw = jnp.where(sw < 0, sw ^ jnp.int32(0x7FFFFFFF), sw)

    maxk = jnp.max(key.reshape(blk, n // 128, 128), axis=1)
    lmax = jnp.max(maxk, axis=1, keepdims=True)
    lmin = jnp.min(maxk, axis=1, keepdims=True)

    def cbody(_, c):
        lo_c, hi_c = c
        mid = (lo_c & hi_c) + ((lo_c ^ hi_c) >> 1)
        cnt = jnp.sum((maxk >= mid).astype(jnp.int32), axis=1, keepdims=True)
        ge = cnt >= _K
        return jnp.where(ge, mid, lo_c), jnp.where(ge, hi_c, mid)

    c64, _ = jax.lax.fori_loop(0, 32, cbody, (lmin, lmax + 1))

    lo0 = c64
    hi0 = jnp.maximum(lmax, keyw) + 1
    cl0 = (jnp.sum((key >= lo0).astype(jnp.int32), axis=1, keepdims=True)
           + (keyw >= lo0).astype(jnp.int32))
    ch0 = jnp.zeros_like(cl0)

    def _done(lo, hi, cl, ch):
        return (hi <= lo + 1) | (cl == _K) | (ch == _K - 1)

    def vcond(c):
        lo, hi, cl, ch = c
        return jnp.any(~_done(lo, hi, cl, ch))

    def vbody(c):
        lo, hi, cl, ch = c
        span = hi - lo
        mid_bis = (lo & hi) + ((lo ^ hi) >> 1)
        bnd = jnp.maximum(span >> 3, 1)
        lo_f = lo.astype(jnp.float32)
        span_f = span.astype(jnp.float32)
        denom = jnp.maximum(cl - ch, 1).astype(jnp.float32)
        mids = []
        for t in _TARGETS:
            frac = jnp.clip((cl - t).astype(jnp.float32) / denom, 0.0, 1.0)
            mid_f = lo_f + frac * span_f
            mid_i = jnp.clip(mid_f.astype(jnp.int32), lo + bnd, hi - bnd)
            mids.append(jnp.where(span > 0, mid_i, mid_bis))
        cnts = [(jnp.sum((key >= m).astype(jnp.int32), axis=1, keepdims=True)
                 + (keyw >= m).astype(jnp.int32)) for m in mids]
        upd = ~_done(lo, hi, cl, ch)
        for m, cnt in zip(mids, cnts):
            ge = (cnt >= _K) & upd & (m > lo) & (m < hi)
            lt = (cnt < _K) & upd & (m > lo) & (m < hi)
            lo = jnp.where(ge, m, lo)
            cl = jnp.where(ge, cnt, cl)
            hi = jnp.where(lt, m, hi)
            ch = jnp.where(lt, cnt, ch)
        return lo, hi, cl, ch

    lo, hi, cl, ch = jax.lax.while_loop(vcond, vbody, (lo0, hi0, cl0, ch0))

    imax = jnp.int32(0x7FFFFFFF)
    imin = jnp.int32(-0x80000000)
    mmin = jnp.min(jnp.where(key >= lo, key, imax), axis=1, keepdims=True)
    mmin = jnp.where(keyw >= lo, jnp.minimum(mmin, keyw), mmin)
    mmax = jnp.max(jnp.where(key < hi, key, imin), axis=1, keepdims=True)
    mmax = jnp.where(keyw < hi, jnp.maximum(mmax, keyw), mmax)

    conv = hi <= lo + 1
    fin_a = cl == _K
    m64 = jnp.where(conv, lo, jnp.where(fin_a, mmin, mmax))

    eqw = keyw == m64
    gtw = keyw > m64
    cnt_eq = (jnp.sum((key == m64).astype(jnp.int32), axis=1, keepdims=True)
              + eqw.astype(jnp.int32))
    cnt_ge = jnp.where(conv, cl, jnp.where(fin_a, _K, ch + cnt_eq))
    cnt_gt = cnt_ge - cnt_eq

    idx = jax.lax.broadcasted_iota(jnp.int32, (blk, n), 1)
    loi0 = jnp.where(cnt_ge == _K, jnp.int32(n), jnp.int32(0))
    hii0 = jnp.full((blk, 1), n + 1, jnp.int32)

    def icond(c):
        lo_i, hi_i = c
        return jnp.any(hi_i - lo_i > 1)

    def ibody(c):
        lo_i, hi_i = c
        mid = lo_i + (hi_i - lo_i) // 2
        f = cnt_gt + jnp.sum(((key == m64) & (idx < mid)).astype(jnp.int32),
                             axis=1, keepdims=True)
        ge = f >= _K
        return jnp.where(ge, lo_i, mid), jnp.where(ge, mid, hi_i)

    _, istar = jax.lax.while_loop(icond, ibody, (loi0, hii0))

    op_ref[...] = (key > m64) | ((key == m64) & (idx < istar))
    wm_ref[...] = gtw | (eqw & (istar == n + 1))


def kernel(slot_scores, write_score):
    b, n = slot_scores.shape
    w2d = write_score.reshape(b, 1)
    grid = b // _BLK
    op_mask, wm2d = pl.pallas_call(
        _select_kernel,
        grid=(grid,),
        in_specs=[
            pl.BlockSpec((_BLK, n), lambda i: (i, 0)),
            pl.BlockSpec((_BLK, 1), lambda i: (i, 0)),
        ],
        out_specs=[
            pl.BlockSpec((_BLK, n), lambda i: (i, 0)),
            pl.BlockSpec((_BLK, 1), lambda i: (i, 0)),
        ],
        out_shape=[
            jax.ShapeDtypeStruct((b, n), jnp.bool_),
            jax.ShapeDtypeStruct((b, 1), jnp.bool_),
        ],
    )(slot_scores, w2d)
    return op_mask, wm2d.reshape(b)
